# baseline (speedup 1.0000x reference)
import jax
import jax.numpy as jnp
from jax import lax
from jax.experimental import pallas as pl
from jax.experimental.pallas import tpu as pltpu
from jax.experimental.pallas import tpu_sc as plsc

_BATCH = 8
_SEQ = 50
_VOCAB = 100000
_ALPHA = 1.0
_TOTAL = _BATCH * _SEQ
_LANES = 16
_NUM_CORES = 2
_ACTIVE_WORKERS = _TOTAL // _LANES


def _gather_body(chosen_hbm, probs_hbm, out_hbm, idx_v, vals_v, sem):
    wid = lax.axis_index("s") * _NUM_CORES + lax.axis_index("c")

    @pl.when(wid < _ACTIVE_WORKERS)
    def _():
        base = wid * _LANES
        pltpu.sync_copy(chosen_hbm.at[pl.ds(base, _LANES)], idx_v)
        tok = idx_v[...]
        pos = lax.iota(jnp.int32, _LANES) + base
        idx_v[...] = pos * _VOCAB + tok
        pltpu.async_copy(probs_hbm.at[idx_v], vals_v, sem).wait()
        pltpu.sync_copy(vals_v, out_hbm.at[pl.ds(base, _LANES)])


def _sc_gather(chosen_flat, probs_flat):
    mesh = plsc.VectorSubcoreMesh(core_axis_name="c", subcore_axis_name="s")
    return pl.kernel(
        _gather_body,
        mesh=mesh,
        out_type=jax.ShapeDtypeStruct((_TOTAL,), jnp.float32),
        scratch_types=[
            pltpu.VMEM((_LANES,), jnp.int32),
            pltpu.VMEM((_LANES,), jnp.float32),
            pltpu.SemaphoreType.DMA,
        ],
    )(chosen_flat, probs_flat)


def _loss_body(p_ref, m_ref, r_ref, out_ref):
    p = p_ref[...]
    m = m_ref[...]
    loss = -jnp.log(p) * m
    s = jnp.sum(loss, axis=1, keepdims=True)
    n = jnp.sum(m, axis=1, keepdims=True)
    out_ref[...] = s * r_ref[...] / n * _ALPHA


def kernel(chosen_tokens, probs, time_step_mask, delta_rewards):
    chosen_flat = chosen_tokens.reshape(_TOTAL)
    probs_flat = probs.reshape(_BATCH * _SEQ * _VOCAB)
    token_probs = _sc_gather(chosen_flat, probs_flat).reshape(_BATCH, _SEQ)
    out = pl.pallas_call(
        _loss_body,
        out_shape=jax.ShapeDtypeStruct((_BATCH, 1), jnp.float32),
    )(token_probs, time_step_mask, delta_rewards.reshape(_BATCH, 1))
    return out.reshape(_BATCH)


# speedup vs baseline: 17.3080x; 17.3080x over previous
import jax
import jax.numpy as jnp
from jax import lax
from jax.experimental import pallas as pl
from jax.experimental.pallas import tpu as pltpu
from jax.experimental.pallas import tpu_sc as plsc

_BATCH = 8
_SEQ = 50
_VOCAB = 100000
_ALPHA = 1.0
_TOTAL = _BATCH * _SEQ
_LANES = 16
_NUM_CORES = 2
_ACTIVE_WORKERS = _TOTAL // _LANES


def _gather_body(chosen_hbm, probs_hbm, out_hbm, tok_v, buf_v, val_v, sem):
    wid = lax.axis_index("s") * _NUM_CORES + lax.axis_index("c")

    @pl.when(wid < _ACTIVE_WORKERS)
    def _():
        base = wid * _LANES
        pltpu.sync_copy(chosen_hbm.at[pl.ds(base, _LANES)], tok_v)
        tok = tok_v[...]
        copies = []
        for j in range(_LANES):
            pos = base + j
            b = (pos * 41) >> 11
            t = pos - b * _SEQ
            v = tok[j]
            v8 = pl.multiple_of(v & ~7, 8)
            copies.append(
                pltpu.async_copy(
                    probs_hbm.at[b, t, pl.ds(v8, 8)],
                    buf_v.at[pl.ds(j * 8, 8)],
                    sem,
                )
            )
        for c in copies:
            c.wait()
        flat_idx = lax.iota(jnp.int32, _LANES) * 8 + (tok & 7)
        val_v[...] = plsc.load_gather(buf_v, [flat_idx])
        pltpu.sync_copy(val_v, out_hbm.at[pl.ds(base, _LANES)])


def _sc_gather(chosen_flat, probs):
    mesh = plsc.VectorSubcoreMesh(core_axis_name="c", subcore_axis_name="s")
    return pl.kernel(
        _gather_body,
        mesh=mesh,
        out_type=jax.ShapeDtypeStruct((_TOTAL,), jnp.float32),
        scratch_types=[
            pltpu.VMEM((_LANES,), jnp.int32),
            pltpu.VMEM((_LANES * 8,), jnp.float32),
            pltpu.VMEM((_LANES,), jnp.float32),
            pltpu.SemaphoreType.DMA,
        ],
        compiler_params=pltpu.CompilerParams(needs_layout_passes=False),
    )(chosen_flat, probs)


def _loss_body(p_ref, m_ref, r_ref, out_ref):
    p = p_ref[...]
    m = m_ref[...]
    loss = -jnp.log(p) * m
    s = jnp.sum(loss, axis=1, keepdims=True)
    n = jnp.sum(m, axis=1, keepdims=True)
    out_ref[...] = s * r_ref[...] / n * _ALPHA


def kernel(chosen_tokens, probs, time_step_mask, delta_rewards):
    chosen_flat = chosen_tokens.reshape(_TOTAL)
    token_probs = _sc_gather(chosen_flat, probs).reshape(_BATCH, _SEQ)
    out = pl.pallas_call(
        _loss_body,
        out_shape=jax.ShapeDtypeStruct((_BATCH, 1), jnp.float32),
    )(token_probs, time_step_mask, delta_rewards.reshape(_BATCH, 1))
    return out.reshape(_BATCH)
